# baseline (speedup 1.0000x reference)
import functools

import jax
import jax.numpy as jnp
from jax import lax
from jax.experimental import pallas as pl
from jax.experimental.pallas import tpu as pltpu
from jax.experimental.pallas import tpu_sc as plsc

_NC = 2
_NS = 16
_NW = _NC * _NS
_CHUNK = 128
_BLK = 2528
_G = 128
_LANES = 16

_DOT = dict(preferred_element_type=jnp.float32, precision=lax.Precision.HIGHEST)


def _mm(a, b):
    return lax.dot_general(a, b, (((1,), (0,)), ((), ())), **_DOT)


def _sc_aggregate(p_pad, edges):
    n_pad, h = p_pad.shape
    cpw, ch = edges.shape[2], edges.shape[3]
    rows_per_tile = n_pad // _NS
    nfull = rows_per_tile // ch
    rem = rows_per_tile % ch
    tile_budget = (2097151 * 4 - 2 * n_pad * h * 4) // _NS
    idx_bytes = 2 * cpw * ch * 4
    nring = max(2, min(6, (tile_budget - idx_bytes - 4096) // (ch * h * 4)))
    mesh = plsc.VectorSubcoreMesh(core_axis_name="c", subcore_axis_name="s")

    def body(p_hbm, edges_hbm, out0, out1,
             src_v, dst_v, rows, acc, p_spm, gsems):
        zbuf = rows[0]
        cid = lax.axis_index("c")
        sid = lax.axis_index("s")
        wid = sid * _NC + cid
        r0 = sid * rows_per_tile

        pltpu.async_copy(p_hbm.at[pl.ds(r0, rows_per_tile)],
                         p_spm.at[pl.ds(r0, rows_per_tile)], gsems[0])

        zv = jnp.zeros((_LANES,), jnp.float32)

        def zrow(r, carry):
            for c in range(h // _LANES):
                zbuf[r, pl.ds(c * _LANES, _LANES)] = zv
            return carry

        lax.fori_loop(0, ch, zrow, 0)
        for k in range(nfull):
            pltpu.sync_copy(zbuf, acc.at[pl.ds(r0 + k * ch, ch)])
        if rem:
            pltpu.sync_copy(zbuf.at[pl.ds(0, rem)],
                            acc.at[pl.ds(r0 + nfull * ch, rem)])

        pltpu.sync_copy(edges_hbm.at[0, wid], src_v)
        pltpu.sync_copy(edges_hbm.at[1, wid], dst_v)
        pltpu.make_async_copy(p_hbm.at[pl.ds(r0, rows_per_tile)],
                              p_spm.at[pl.ds(r0, rows_per_tile)],
                              gsems[0]).wait()

        plsc.subcore_barrier()

        def gdesc(c, b):
            src = p_hbm if b == 0 else p_spm
            return pltpu.make_async_copy(src.at[src_v.at[c]], rows[b],
                                         gsems[b])

        for b in range(min(nring, cpw)):
            gdesc(b, b).start()

        def handle(jj, b):
            gdesc(jj, b).wait()
            pltpu.sync_copy(rows[b], acc.at[dst_v.at[jj]], add=True)

            @pl.when(jj + nring < cpw)
            def _():
                gdesc(jj + nring, b).start()

        def step(t, carry):
            for b in range(nring):
                handle(t * nring + b, b)
            return carry

        main = cpw // nring
        lax.fori_loop(0, main, step, 0)
        for b in range(cpw - main * nring):
            handle(main * nring + b, b)

        plsc.subcore_barrier()

        @pl.when(cid == 0)
        def _():
            pltpu.sync_copy(acc.at[pl.ds(r0, rows_per_tile)],
                            out0.at[pl.ds(r0, rows_per_tile)])

        @pl.when(cid == 1)
        def _():
            pltpu.sync_copy(acc.at[pl.ds(r0, rows_per_tile)],
                            out1.at[pl.ds(r0, rows_per_tile)])

    fn = pl.kernel(
        body,
        out_type=(jax.ShapeDtypeStruct((n_pad, h), jnp.float32),
                  jax.ShapeDtypeStruct((n_pad, h), jnp.float32)),
        mesh=mesh,
        scratch_types=[
            pltpu.VMEM((cpw, ch), jnp.int32),
            pltpu.VMEM((cpw, ch), jnp.int32),
            [pltpu.VMEM((ch, h), jnp.float32)] * nring,
            pltpu.VMEM_SHARED((n_pad, h), jnp.float32),
            pltpu.VMEM_SHARED((n_pad, h), jnp.float32),
            [pltpu.SemaphoreType.DMA] * nring,
        ],
        compiler_params=pltpu.CompilerParams(use_tc_tiling_on_sc=False),
    )
    return fn(p_pad, edges)


def _rmask(limit, blk_rows, blk):
    i = pl.program_id(0)
    rows = i * blk + lax.broadcasted_iota(jnp.int32, (blk_rows, 1), 0)
    return (rows < limit).astype(jnp.float32)


def _proj_body(n2, blk2, x_ref, w_ref, o_ref):
    xv = jnp.where(_rmask(n2, x_ref.shape[0], blk2) > 0.0, x_ref[...], 0.0)
    o_ref[...] = _mm(xv, w_ref[...])


def _stats_body(n2, blk2, p_ref, a0_ref, a1_ref, b1_ref, w2_ref, b2_ref,
                v_ref, st_ref):
    i = pl.program_id(0)
    u = jnp.maximum(p_ref[...] + a0_ref[...] + a1_ref[...] + b1_ref[...], 0.0)
    v = _mm(u, w2_ref[...]) + b2_ref[...]
    v_ref[...] = v
    vm = v * _rmask(n2, v.shape[0], blk2)
    s = jnp.concatenate([jnp.sum(vm, axis=0, keepdims=True),
                         jnp.sum(vm * vm, axis=0, keepdims=True)], axis=0)

    @pl.when(i == 0)
    def _():
        st_ref[...] = jnp.zeros_like(st_ref)

    st_ref[...] += s


def _bnorm(n, n2, blk2, h, v_ref, st_ref, g_ref, be_ref):
    s = st_ref[...]
    fold = lambda r: r[:, :h] + r[:, h:]
    mean = fold(s[0:1, :]) / n
    var = fold(s[1:2, :]) / n - mean * mean
    inv = lax.rsqrt(var + 1e-5)
    mean2 = jnp.concatenate([mean, mean], axis=1)
    inv2 = jnp.concatenate([inv, inv], axis=1)
    hh = jnp.maximum((v_ref[...] - mean2) * inv2 * g_ref[...] + be_ref[...],
                     0.0)
    return hh * _rmask(n2, v_ref.shape[0], blk2)


def _norm_proj_body(n, n2, blk2, h, v_ref, st_ref, g_ref, be_ref, w1n_ref,
                    o_ref):
    o_ref[...] = _mm(_bnorm(n, n2, blk2, h, v_ref, st_ref, g_ref, be_ref),
                     w1n_ref[...])


def _pool_head_body(n, n2, blk2, h, nb, v_ref, st_ref, g_ref, be_ref,
                    be_e_ref, be_o_ref, wh1_ref, bh1_ref, wh2_ref, bh2_ref,
                    o_ref, hg_ref):
    i = pl.program_id(0)
    hh = _bnorm(n, n2, blk2, h, v_ref, st_ref, g_ref, be_ref)
    gi = lax.broadcasted_iota(jnp.int32, (_G, hh.shape[0]), 0)
    oe = (gi == be_e_ref[0]).astype(jnp.float32)
    oo = (gi == be_o_ref[0]).astype(jnp.float32)
    contrib = _mm(oe, hh)[:, :h] + _mm(oo, hh)[:, h:]

    @pl.when(i == 0)
    def _():
        hg_ref[...] = contrib

    @pl.when(i > 0)
    def _():
        hg_ref[...] += contrib

    @pl.when(i == nb - 1)
    def _():
        t = jnp.maximum(_mm(hg_ref[...], wh1_ref[...]) + bh1_ref[...], 0.0)
        o_ref[...] = _mm(t, wh2_ref[...]) + bh2_ref[...]


def kernel(x, edge_index, batch,
           W1_0, b1_0, W2_0, b2_0, g_0, be_0,
           W1_1, b1_1, W2_1, b2_1, g_1, be_1,
           W1_2, b1_2, W2_2, b2_2, g_2, be_2,
           Wh1, bh1, Wh2, bh2):
    n, d = x.shape
    h = W1_0.shape[1]
    e = edge_index.shape[1]
    c = Wh2.shape[1]

    n_pad = -(-(n + 1) // (_NS * 8)) * (_NS * 8)

    epw, e_rem = divmod(e, _NW)
    ch = next((cc for cc in range(_CHUNK, 7, -8)
               if e_rem == 0 and epw % cc == 0), None)
    if ch is not None:
        edges = edge_index.reshape(2, _NW, epw // ch, ch)
    else:
        ch = _CHUNK
        cpw = -(-e // (_NW * ch))
        npe = _NW * cpw * ch - e
        src_pad = jnp.full((npe,), n, jnp.int32)
        dst_pad = (jnp.arange(npe, dtype=jnp.int32) % (n_pad - n)) + n
        edges = jnp.concatenate(
            [edge_index, jnp.stack([src_pad, dst_pad])],
            axis=1).reshape(2, _NW, cpw, ch)
    n2 = n // 2
    n_pad2 = n_pad // 2
    h2 = 2 * h
    blk2 = _BLK // 2
    nb = n_pad // _BLK
    bp = jnp.pad(batch, (0, n_pad - n), constant_values=-1).reshape(n_pad2, 2)
    batch_e = bp[:, 0].reshape(nb, 1, blk2)
    batch_o = bp[:, 1].reshape(nb, 1, blk2)

    row = lambda a: a.reshape(1, -1)
    tile2 = lambda a: jnp.concatenate([a, a], axis=-1).reshape(1, -1)

    def bdiag(w):
        z = jnp.zeros_like(w)
        return jnp.concatenate(
            [jnp.concatenate([w, z], axis=1),
             jnp.concatenate([z, w], axis=1)], axis=0)

    ws = {
        0: (tile2(b1_0), bdiag(W2_0), tile2(b2_0), tile2(g_0), tile2(be_0)),
        1: (tile2(b1_1), bdiag(W2_1), tile2(b2_1), tile2(g_1), tile2(be_1)),
        2: (tile2(b1_2), bdiag(W2_2), tile2(b2_2), tile2(g_2), tile2(be_2)),
    }

    r2spec = pl.BlockSpec((blk2, h2), lambda i: (i, 0))

    def full(s):
        return pl.BlockSpec(s, lambda i: (0,) * len(s))

    f32 = jnp.float32

    p2 = pl.pallas_call(
        functools.partial(_proj_body, n2, blk2),
        grid=(nb,),
        in_specs=[pl.BlockSpec((blk2, 2 * d), lambda i: (i, 0)),
                  full((2 * d, h2))],
        out_specs=r2spec,
        out_shape=jax.ShapeDtypeStruct((n_pad2, h2), f32),
    )(x.reshape(n2, 2 * d), bdiag(W1_0))

    for i in range(3):
        a0, a1 = _sc_aggregate(p2.reshape(n_pad, h), edges)
        a02 = a0.reshape(n_pad2, h2)
        a12 = a1.reshape(n_pad2, h2)
        b1t, W2bd, b2t, gt, bet = ws[i]
        v2, st = pl.pallas_call(
            functools.partial(_stats_body, n2, blk2),
            grid=(nb,),
            in_specs=[r2spec, r2spec, r2spec, full((1, h2)), full((h2, h2)),
                      full((1, h2))],
            out_specs=[r2spec, full((2, h2))],
            out_shape=[jax.ShapeDtypeStruct((n_pad2, h2), f32),
                       jax.ShapeDtypeStruct((2, h2), f32)],
        )(p2, a02, a12, b1t, W2bd, b2t)
        if i < 2:
            w1nbd = bdiag(W1_1 if i == 0 else W1_2)
            p2 = pl.pallas_call(
                functools.partial(_norm_proj_body, n, n2, blk2, h),
                grid=(nb,),
                in_specs=[r2spec, full((2, h2)), full((1, h2)),
                          full((1, h2)), full((h2, h2))],
                out_specs=r2spec,
                out_shape=jax.ShapeDtypeStruct((n_pad2, h2), f32),
            )(v2, st, gt, bet, w1nbd)
        else:
            out = pl.pallas_call(
                functools.partial(_pool_head_body, n, n2, blk2, h, nb),
                grid=(nb,),
                in_specs=[r2spec, full((2, h2)), full((1, h2)),
                          full((1, h2)),
                          pl.BlockSpec((1, 1, blk2), lambda i: (i, 0, 0)),
                          pl.BlockSpec((1, 1, blk2), lambda i: (i, 0, 0)),
                          full((h, h)), full((1, h)), full((h, c)),
                          full((1, c))],
                out_specs=full((_G, c)),
                out_shape=jax.ShapeDtypeStruct((_G, c), f32),
                scratch_shapes=[pltpu.VMEM((_G, h), f32)],
            )(v2, st, gt, bet, batch_e, batch_o, Wh1, row(bh1), Wh2,
              row(bh2))
    return out


# speedup vs baseline: 1.0388x; 1.0388x over previous
import functools

import jax
import jax.numpy as jnp
from jax import lax
from jax.experimental import pallas as pl
from jax.experimental.pallas import tpu as pltpu
from jax.experimental.pallas import tpu_sc as plsc

_NC = 2
_NS = 16
_NW = _NC * _NS
_CHUNK = 128
_BLK = 2528
_G = 128
_LANES = 16

_DOT = dict(preferred_element_type=jnp.float32, precision=lax.Precision.HIGHEST)


def _mm(a, b):
    return lax.dot_general(a, b, (((1,), (0,)), ((), ())), **_DOT)


def _sc_aggregate(p_pad, edges):
    n_pad, h = p_pad.shape
    cpw, ch = edges.shape[2], edges.shape[3]
    rows_per_tile = n_pad // _NS
    nfull = rows_per_tile // ch
    rem = rows_per_tile % ch
    tile_budget = (2097151 * 4 - 2 * n_pad * h * 4) // _NS
    idx_bytes = 2 * cpw * ch * 4
    nring = max(2, min(6, (tile_budget - idx_bytes - 4096) // (ch * h * 4)))
    mesh = plsc.VectorSubcoreMesh(core_axis_name="c", subcore_axis_name="s")

    def body(p_hbm, edges_hbm, out0, out1,
             src_v, dst_v, rows, acc, p_spm, gsems):
        zbuf = rows[0]
        cid = lax.axis_index("c")
        sid = lax.axis_index("s")
        wid = sid * _NC + cid
        r0 = sid * rows_per_tile

        pltpu.async_copy(p_hbm.at[pl.ds(r0, rows_per_tile)],
                         p_spm.at[pl.ds(r0, rows_per_tile)], gsems[0])

        zv = jnp.zeros((_LANES,), jnp.float32)

        def zrow(r, carry):
            for c in range(h // _LANES):
                zbuf[r, pl.ds(c * _LANES, _LANES)] = zv
            return carry

        lax.fori_loop(0, ch, zrow, 0)
        for k in range(nfull):
            pltpu.sync_copy(zbuf, acc.at[pl.ds(r0 + k * ch, ch)])
        if rem:
            pltpu.sync_copy(zbuf.at[pl.ds(0, rem)],
                            acc.at[pl.ds(r0 + nfull * ch, rem)])

        pltpu.sync_copy(edges_hbm.at[0, wid], src_v)
        pltpu.sync_copy(edges_hbm.at[1, wid], dst_v)
        pltpu.make_async_copy(p_hbm.at[pl.ds(r0, rows_per_tile)],
                              p_spm.at[pl.ds(r0, rows_per_tile)],
                              gsems[0]).wait()

        plsc.subcore_barrier()

        def gdesc(c, b):
            return pltpu.make_async_copy(p_spm.at[src_v.at[c]], rows[b],
                                         gsems[b])

        for b in range(min(nring, cpw)):
            gdesc(b, b).start()

        def handle(jj, b):
            gdesc(jj, b).wait()
            pltpu.sync_copy(rows[b], acc.at[dst_v.at[jj]], add=True)

            @pl.when(jj + nring < cpw)
            def _():
                gdesc(jj + nring, b).start()

        def step(t, carry):
            for b in range(nring):
                handle(t * nring + b, b)
            return carry

        main = cpw // nring
        lax.fori_loop(0, main, step, 0)
        for b in range(cpw - main * nring):
            handle(main * nring + b, b)

        plsc.subcore_barrier()

        @pl.when(cid == 0)
        def _():
            pltpu.sync_copy(acc.at[pl.ds(r0, rows_per_tile)],
                            out0.at[pl.ds(r0, rows_per_tile)])

        @pl.when(cid == 1)
        def _():
            pltpu.sync_copy(acc.at[pl.ds(r0, rows_per_tile)],
                            out1.at[pl.ds(r0, rows_per_tile)])

    fn = pl.kernel(
        body,
        out_type=(jax.ShapeDtypeStruct((n_pad, h), jnp.float32),
                  jax.ShapeDtypeStruct((n_pad, h), jnp.float32)),
        mesh=mesh,
        scratch_types=[
            pltpu.VMEM((cpw, ch), jnp.int32),
            pltpu.VMEM((cpw, ch), jnp.int32),
            [pltpu.VMEM((ch, h), jnp.float32)] * nring,
            pltpu.VMEM_SHARED((n_pad, h), jnp.float32),
            pltpu.VMEM_SHARED((n_pad, h), jnp.float32),
            [pltpu.SemaphoreType.DMA] * nring,
        ],
        compiler_params=pltpu.CompilerParams(use_tc_tiling_on_sc=False),
    )
    return fn(p_pad, edges)


def _rmask(limit, blk_rows, blk):
    i = pl.program_id(0)
    rows = i * blk + lax.broadcasted_iota(jnp.int32, (blk_rows, 1), 0)
    return (rows < limit).astype(jnp.float32)


def _proj_body(n2, blk2, x_ref, w_ref, o_ref):
    xv = jnp.where(_rmask(n2, x_ref.shape[0], blk2) > 0.0, x_ref[...], 0.0)
    o_ref[...] = _mm(xv, w_ref[...])


def _stats_body(n2, blk2, p_ref, a0_ref, a1_ref, b1_ref, w2_ref, b2_ref,
                v_ref, st_ref):
    i = pl.program_id(0)
    u = jnp.maximum(p_ref[...] + a0_ref[...] + a1_ref[...] + b1_ref[...], 0.0)
    v = _mm(u, w2_ref[...]) + b2_ref[...]
    v_ref[...] = v
    vm = v * _rmask(n2, v.shape[0], blk2)
    s = jnp.concatenate([jnp.sum(vm, axis=0, keepdims=True),
                         jnp.sum(vm * vm, axis=0, keepdims=True)], axis=0)

    @pl.when(i == 0)
    def _():
        st_ref[...] = jnp.zeros_like(st_ref)

    st_ref[...] += s


def _bnorm(n, n2, blk2, h, v_ref, st_ref, g_ref, be_ref):
    s = st_ref[...]
    fold = lambda r: r[:, :h] + r[:, h:]
    mean = fold(s[0:1, :]) / n
    var = fold(s[1:2, :]) / n - mean * mean
    inv = lax.rsqrt(var + 1e-5)
    mean2 = jnp.concatenate([mean, mean], axis=1)
    inv2 = jnp.concatenate([inv, inv], axis=1)
    hh = jnp.maximum((v_ref[...] - mean2) * inv2 * g_ref[...] + be_ref[...],
                     0.0)
    return hh * _rmask(n2, v_ref.shape[0], blk2)


def _norm_proj_body(n, n2, blk2, h, v_ref, st_ref, g_ref, be_ref, w1n_ref,
                    o_ref):
    o_ref[...] = _mm(_bnorm(n, n2, blk2, h, v_ref, st_ref, g_ref, be_ref),
                     w1n_ref[...])


def _pool_head_body(n, n2, blk2, h, nb, v_ref, st_ref, g_ref, be_ref,
                    be_e_ref, be_o_ref, wh1_ref, bh1_ref, wh2_ref, bh2_ref,
                    o_ref, hg_ref):
    i = pl.program_id(0)
    hh = _bnorm(n, n2, blk2, h, v_ref, st_ref, g_ref, be_ref)
    gi = lax.broadcasted_iota(jnp.int32, (_G, hh.shape[0]), 0)
    oe = (gi == be_e_ref[0]).astype(jnp.float32)
    oo = (gi == be_o_ref[0]).astype(jnp.float32)
    contrib = _mm(oe, hh)[:, :h] + _mm(oo, hh)[:, h:]

    @pl.when(i == 0)
    def _():
        hg_ref[...] = contrib

    @pl.when(i > 0)
    def _():
        hg_ref[...] += contrib

    @pl.when(i == nb - 1)
    def _():
        t = jnp.maximum(_mm(hg_ref[...], wh1_ref[...]) + bh1_ref[...], 0.0)
        o_ref[...] = _mm(t, wh2_ref[...]) + bh2_ref[...]


def kernel(x, edge_index, batch,
           W1_0, b1_0, W2_0, b2_0, g_0, be_0,
           W1_1, b1_1, W2_1, b2_1, g_1, be_1,
           W1_2, b1_2, W2_2, b2_2, g_2, be_2,
           Wh1, bh1, Wh2, bh2):
    n, d = x.shape
    h = W1_0.shape[1]
    e = edge_index.shape[1]
    c = Wh2.shape[1]

    n_pad = -(-(n + 1) // (_NS * 8)) * (_NS * 8)

    epw, e_rem = divmod(e, _NW)
    ch = next((cc for cc in range(_CHUNK, 7, -8)
               if e_rem == 0 and epw % cc == 0), None)
    if ch is not None:
        edges = edge_index.reshape(2, _NW, epw // ch, ch)
    else:
        ch = _CHUNK
        cpw = -(-e // (_NW * ch))
        npe = _NW * cpw * ch - e
        src_pad = jnp.full((npe,), n, jnp.int32)
        dst_pad = (jnp.arange(npe, dtype=jnp.int32) % (n_pad - n)) + n
        edges = jnp.concatenate(
            [edge_index, jnp.stack([src_pad, dst_pad])],
            axis=1).reshape(2, _NW, cpw, ch)
    n2 = n // 2
    n_pad2 = n_pad // 2
    h2 = 2 * h
    blk2 = _BLK // 2
    nb = n_pad // _BLK
    bp = jnp.pad(batch, (0, n_pad - n), constant_values=-1).reshape(n_pad2, 2)
    batch_e = bp[:, 0].reshape(nb, 1, blk2)
    batch_o = bp[:, 1].reshape(nb, 1, blk2)

    row = lambda a: a.reshape(1, -1)
    tile2 = lambda a: jnp.concatenate([a, a], axis=-1).reshape(1, -1)

    def bdiag(w):
        z = jnp.zeros_like(w)
        return jnp.concatenate(
            [jnp.concatenate([w, z], axis=1),
             jnp.concatenate([z, w], axis=1)], axis=0)

    ws = {
        0: (tile2(b1_0), bdiag(W2_0), tile2(b2_0), tile2(g_0), tile2(be_0)),
        1: (tile2(b1_1), bdiag(W2_1), tile2(b2_1), tile2(g_1), tile2(be_1)),
        2: (tile2(b1_2), bdiag(W2_2), tile2(b2_2), tile2(g_2), tile2(be_2)),
    }

    r2spec = pl.BlockSpec((blk2, h2), lambda i: (i, 0))

    def full(s):
        return pl.BlockSpec(s, lambda i: (0,) * len(s))

    f32 = jnp.float32

    p2 = pl.pallas_call(
        functools.partial(_proj_body, n2, blk2),
        grid=(nb,),
        in_specs=[pl.BlockSpec((blk2, 2 * d), lambda i: (i, 0)),
                  full((2 * d, h2))],
        out_specs=r2spec,
        out_shape=jax.ShapeDtypeStruct((n_pad2, h2), f32),
    )(x.reshape(n2, 2 * d), bdiag(W1_0))

    for i in range(3):
        a0, a1 = _sc_aggregate(p2.reshape(n_pad, h), edges)
        a02 = a0.reshape(n_pad2, h2)
        a12 = a1.reshape(n_pad2, h2)
        b1t, W2bd, b2t, gt, bet = ws[i]
        v2, st = pl.pallas_call(
            functools.partial(_stats_body, n2, blk2),
            grid=(nb,),
            in_specs=[r2spec, r2spec, r2spec, full((1, h2)), full((h2, h2)),
                      full((1, h2))],
            out_specs=[r2spec, full((2, h2))],
            out_shape=[jax.ShapeDtypeStruct((n_pad2, h2), f32),
                       jax.ShapeDtypeStruct((2, h2), f32)],
        )(p2, a02, a12, b1t, W2bd, b2t)
        if i < 2:
            w1nbd = bdiag(W1_1 if i == 0 else W1_2)
            p2 = pl.pallas_call(
                functools.partial(_norm_proj_body, n, n2, blk2, h),
                grid=(nb,),
                in_specs=[r2spec, full((2, h2)), full((1, h2)),
                          full((1, h2)), full((h2, h2))],
                out_specs=r2spec,
                out_shape=jax.ShapeDtypeStruct((n_pad2, h2), f32),
            )(v2, st, gt, bet, w1nbd)
        else:
            out = pl.pallas_call(
                functools.partial(_pool_head_body, n, n2, blk2, h, nb),
                grid=(nb,),
                in_specs=[r2spec, full((2, h2)), full((1, h2)),
                          full((1, h2)),
                          pl.BlockSpec((1, 1, blk2), lambda i: (i, 0, 0)),
                          pl.BlockSpec((1, 1, blk2), lambda i: (i, 0, 0)),
                          full((h, h)), full((1, h)), full((h, c)),
                          full((1, c))],
                out_specs=full((_G, c)),
                out_shape=jax.ShapeDtypeStruct((_G, c), f32),
                scratch_shapes=[pltpu.VMEM((_G, h), f32)],
            )(v2, st, gt, bet, batch_e, batch_o, Wh1, row(bh1), Wh2,
              row(bh2))
    return out
